# Initial kernel scaffold; baseline (speedup 1.0000x reference)
import jax
import jax.numpy as jnp
from jax.experimental import pallas as pl


def kernel(vectors):
    raise NotImplementedError("write your pallas kernel here")


# speedup vs baseline: 26.8587x; 26.8587x over previous
import functools

import jax
import jax.numpy as jnp
from jax import lax
from jax.experimental import pallas as pl
from jax.experimental.pallas import tpu as pltpu
from jax.experimental.pallas import tpu_sc as plsc

_N = 256
_KTOP = 16
_STEEP = 10.0
_B = 128
_NW = 32
_SPW = _B // _NW
_WORDS = _SPW * _N
_ITERS = _WORDS // 16


def _layer_params():
    out = []
    k = 2
    while k <= _N:
        j = k // 2
        while j >= 1:
            out.append((j, k))
            j //= 2
        k *= 2
    return out


_LAYERS = _layer_params()
_NLAYER = len(_LAYERS)

_GATHER_DNUMS = lax.GatherDimensionNumbers(
    offset_dims=(), collapsed_slice_dims=(0,), start_index_map=(0,))


def _lane_shuffle(vec, idx):
    return lax.gather(
        vec, idx[:, None], _GATHER_DNUMS, slice_sizes=(1,),
        mode=lax.GatherScatterMode.PROMISE_IN_BOUNDS)

_ATAN_COEF = (
    0.31830986, -0.10610115, 0.06361912, -0.04508647,
    0.03344584, -0.02307094, 0.01270042, -0.00460235, 0.00078567,
)


def _atan_over_pi(d):
    a = jnp.abs(d)
    inv = jnp.float32(1.0) / jnp.maximum(a, jnp.float32(1e-30))
    r = jnp.minimum(a, inv)
    z = r * r
    p = jnp.full_like(r, jnp.float32(_ATAN_COEF[-1]))
    for c in _ATAN_COEF[-2::-1]:
        p = p * z + jnp.float32(c)
    p = p * r
    res = jnp.where(a > jnp.float32(1.0), jnp.float32(0.5) - p, p)
    return jnp.where(d < jnp.float32(0.0), -res, res)


@functools.partial(
    pl.kernel,
    mesh=plsc.VectorSubcoreMesh(core_axis_name="c", subcore_axis_name="s"),
    out_type=jax.ShapeDtypeStruct((_B * _N,), jnp.float32),
    scratch_types=[
        pltpu.VMEM((_WORDS,), jnp.float32),
        pltpu.VMEM((_WORDS,), jnp.float32),
        pltpu.VMEM((_NLAYER * _WORDS,), jnp.float32),
        pltpu.VMEM((_WORDS,), jnp.float32),
        pltpu.VMEM((_WORDS,), jnp.float32),
    ],
)
def _sc_topk(vec_hbm, out_hbm, xa, xb, al, va, vb):
    wid = lax.axis_index("s") * 2 + lax.axis_index("c")
    base = wid * _WORDS
    pltpu.sync_copy(vec_hbm.at[pl.ds(base, _WORDS)], xa)
    iota = lax.broadcasted_iota(jnp.int32, (16,), 0)

    def fwd_layer(lyr, j, k, src, dst):
        lj = j.bit_length() - 1
        lk = k.bit_length() - 1
        abase = lyr * _WORDS

        def body(w, carry):
            off = w * 16
            i = off + iota
            xi = src[pl.ds(off, 16)]
            if j >= 16:
                xp = src[pl.ds(off ^ j, 16)]
            else:
                xp = _lane_shuffle(xi, iota ^ j)
            im = i & (_N - 1)
            bit = ((im >> lj) ^ (im >> lk)) & 1
            sgn = (1 - 2 * bit).astype(jnp.float32)
            t = _atan_over_pi((xp - xi) * jnp.float32(_STEEP))
            alpha = jnp.float32(0.5) + sgn * t
            dst[pl.ds(off, 16)] = xp + alpha * (xi - xp)
            al[pl.ds(abase + off, 16)] = alpha
            return carry

        lax.fori_loop(0, _ITERS, body, 0)

    xbufs = (xa, xb)
    for lyr, (j, k) in enumerate(_LAYERS):
        fwd_layer(lyr, j, k, xbufs[lyr % 2], xbufs[1 - lyr % 2])

    def vinit(w, carry):
        off = w * 16
        im = (off + iota) & (_N - 1)
        va[pl.ds(off, 16)] = jnp.where(
            im >= _N - _KTOP, jnp.float32(1.0), jnp.float32(0.0))
        return carry

    lax.fori_loop(0, _ITERS, vinit, 0)

    vbufs = (va, vb)
    for r in range(_NLAYER):
        lyr = _NLAYER - 1 - r
        j, _k = _LAYERS[lyr]
        src, dst = vbufs[r % 2], vbufs[1 - r % 2]
        abase = lyr * _WORDS

        def body(w, carry, j=j, src=src, dst=dst, abase=abase):
            off = w * 16
            vi = src[pl.ds(off, 16)]
            if j >= 16:
                vp = src[pl.ds(off ^ j, 16)]
            else:
                vp = _lane_shuffle(vi, iota ^ j)
            a = al[pl.ds(abase + off, 16)]
            dst[pl.ds(off, 16)] = vp + a * (vi - vp)
            return carry

        lax.fori_loop(0, _ITERS, body, 0)

    pltpu.sync_copy(vbufs[_NLAYER % 2], out_hbm.at[pl.ds(base, _WORDS)])


def kernel(vectors):
    out = _sc_topk(vectors.reshape(-1))
    return out.reshape(vectors.shape)


# speedup vs baseline: 37.2989x; 1.3887x over previous
import functools

import jax
import jax.numpy as jnp
from jax import lax
from jax.experimental import pallas as pl
from jax.experimental.pallas import tpu as pltpu
from jax.experimental.pallas import tpu_sc as plsc

_N = 256
_KTOP = 16
_STEEP = 10.0
_B = 128
_NW = 32
_SPW = _B // _NW
_WORDS = _SPW * _N
_CHUNKS = _WORDS // 16


def _phases():
    out = []
    lyr = 0
    k = 2
    while k <= _N:
        js = []
        j = k // 2
        while j >= 1:
            js.append((j, lyr))
            lyr += 1
            j //= 2
        out.append((k, js))
        k *= 2
    return out


_PHASES = _phases()
_NLAYER = sum(len(js) for _, js in _PHASES)

_ATAN_COEF = (0.31827129, -0.10517136, 0.05742714, -0.02718631, 0.0066628)

_GATHER_DNUMS = lax.GatherDimensionNumbers(
    offset_dims=(), collapsed_slice_dims=(0,), start_index_map=(0,))


def _lane_shuffle(vec, idx):
    return lax.gather(
        vec, idx[:, None], _GATHER_DNUMS, slice_sizes=(1,),
        mode=lax.GatherScatterMode.PROMISE_IN_BOUNDS)


def _atan_over_pi(d):
    a = jnp.abs(d)
    inv = jnp.float32(1.0) / jnp.maximum(a, jnp.float32(1e-30))
    r = jnp.minimum(a, inv)
    z = r * r
    p = jnp.full_like(r, jnp.float32(_ATAN_COEF[-1]))
    for c in _ATAN_COEF[-2::-1]:
        p = p * z + jnp.float32(c)
    p = p * r
    res = jnp.where(a > jnp.float32(1.0), jnp.float32(0.5) - p, p)
    return jnp.where(d < jnp.float32(0.0), -res, res)


def _loop(n):
    def deco(body):
        lax.fori_loop(0, n, lambda i, c: (body(i), c)[1], 0)
    return deco


def _pair_index(p, lb):
    return ((p >> lb) << (lb + 1)) | (p & ((1 << lb) - 1))


@functools.partial(
    pl.kernel,
    mesh=plsc.VectorSubcoreMesh(core_axis_name="c", subcore_axis_name="s"),
    out_type=jax.ShapeDtypeStruct((_B * _N,), jnp.float32),
    scratch_types=[
        pltpu.VMEM((_WORDS,), jnp.float32),
        pltpu.VMEM((_NLAYER * _WORDS,), jnp.float32),
        pltpu.VMEM((_WORDS,), jnp.float32),
    ],
)
def _sc_topk(vec_hbm, out_hbm, xs, al, vs):
    wid = lax.axis_index("s") * 2 + lax.axis_index("c")
    base = wid * _WORDS
    pltpu.sync_copy(vec_hbm.at[pl.ds(base, _WORDS)], xs)
    lane = lax.broadcasted_iota(jnp.int32, (16,), 0)
    steep = jnp.float32(_STEEP)
    half = jnp.float32(0.5)

    def lane_sign(lj, lk):
        bits = (lane >> lj) & 1
        if lk is not None:
            bits = bits ^ ((lane >> lk) & 1)
        return (1 - 2 * bits).astype(jnp.float32)

    def chunk_sign(w, lk):
        return (1 - 2 * ((w >> (lk - 4)) & 1)).astype(jnp.float32)

    for k, js in _PHASES:
        lk = k.bit_length() - 1
        pair_js = [(j, lyr) for j, lyr in js if j >= 16]
        grp_js = [(j, lyr) for j, lyr in js if j < 16]

        for j, lyr in pair_js:
            lj = j.bit_length() - 1
            lb = lj - 4
            abase = lyr * _WORDS

            @_loop(_CHUNKS // 2)
            def _(p, j=j, lb=lb, lk=lk, abase=abase):
                w = _pair_index(p, lb)
                off = w * 16
                poff = off ^ j
                xi = xs[pl.ds(off, 16)]
                xp = xs[pl.ds(poff, 16)]
                t = _atan_over_pi((xp - xi) * steep)
                alpha = half + chunk_sign(w & 15, lk) * t
                xs[pl.ds(off, 16)] = xp + alpha * (xi - xp)
                xs[pl.ds(poff, 16)] = xi + alpha * (xp - xi)
                al[pl.ds(abase + off, 16)] = alpha
                al[pl.ds(abase + poff, 16)] = alpha

        if grp_js:
            signs = [
                lane_sign(j.bit_length() - 1, lk if lk <= 3 else None)
                for j, _l in grp_js
            ]

            @_loop(_CHUNKS)
            def _(w, grp_js=grp_js, signs=signs, lk=lk):
                off = w * 16
                xv = xs[pl.ds(off, 16)]
                if lk > 3:
                    cs = chunk_sign(w & 15, lk)
                for (j, lyr), sgn in zip(grp_js, signs):
                    xq = _lane_shuffle(xv, lane ^ j)
                    t = _atan_over_pi((xq - xv) * steep)
                    if lk > 3:
                        alpha = half + cs * (sgn * t)
                    else:
                        alpha = half + sgn * t
                    al[pl.ds(lyr * _WORDS + off, 16)] = alpha
                    xv = xq + alpha * (xv - xq)
                xs[pl.ds(off, 16)] = xv

    @_loop(_CHUNKS)
    def _(w):
        off = w * 16
        im = (off + lane) & (_N - 1)
        vs[pl.ds(off, 16)] = jnp.where(
            im >= _N - _KTOP, jnp.float32(1.0), jnp.float32(0.0))

    for k, js in reversed(_PHASES):
        pair_js = [(j, lyr) for j, lyr in js if j >= 16]
        grp_js = [(j, lyr) for j, lyr in js if j < 16]

        if grp_js:
            @_loop(_CHUNKS)
            def _(w, grp_js=grp_js):
                off = w * 16
                vv = vs[pl.ds(off, 16)]
                for j, lyr in reversed(grp_js):
                    a = al[pl.ds(lyr * _WORDS + off, 16)]
                    vq = _lane_shuffle(vv, lane ^ j)
                    vv = vq + a * (vv - vq)
                vs[pl.ds(off, 16)] = vv

        for j, lyr in reversed(pair_js):
            lb = j.bit_length() - 5
            abase = lyr * _WORDS

            @_loop(_CHUNKS // 2)
            def _(p, j=j, lb=lb, abase=abase):
                w = _pair_index(p, lb)
                off = w * 16
                poff = off ^ j
                vi = vs[pl.ds(off, 16)]
                vp = vs[pl.ds(poff, 16)]
                a = al[pl.ds(abase + off, 16)]
                vs[pl.ds(off, 16)] = vp + a * (vi - vp)
                vs[pl.ds(poff, 16)] = vi + a * (vp - vi)

    pltpu.sync_copy(vs, out_hbm.at[pl.ds(base, _WORDS)])


def kernel(vectors):
    out = _sc_topk(vectors.reshape(-1))
    return out.reshape(vectors.shape)


# speedup vs baseline: 38.5721x; 1.0341x over previous
import functools

import jax
import jax.numpy as jnp
from jax import lax
from jax.experimental import pallas as pl
from jax.experimental.pallas import tpu as pltpu
from jax.experimental.pallas import tpu_sc as plsc

_N = 256
_KTOP = 16
_STEEP = 10.0
_B = 128
_NW = 32
_SPW = _B // _NW
_WORDS = _SPW * _N
_CHUNKS = _WORDS // 16


def _phases():
    out = []
    lyr = 0
    k = 2
    while k <= _N:
        js = []
        j = k // 2
        while j >= 1:
            js.append((j, lyr))
            lyr += 1
            j //= 2
        out.append((k, js))
        k *= 2
    return out


_PHASES = _phases()
_NLAYER = sum(len(js) for _, js in _PHASES)

_ATAN_COEF = (0.31827129, -0.10517136, 0.05742714, -0.02718631, 0.0066628)

_GATHER_DNUMS = lax.GatherDimensionNumbers(
    offset_dims=(), collapsed_slice_dims=(0,), start_index_map=(0,))


def _lane_shuffle(vec, idx):
    return lax.gather(
        vec, idx[:, None], _GATHER_DNUMS, slice_sizes=(1,),
        mode=lax.GatherScatterMode.PROMISE_IN_BOUNDS)


def _atan_over_pi(d):
    a = jnp.abs(d)
    inv = jnp.float32(1.0) / jnp.maximum(a, jnp.float32(1e-30))
    r = jnp.minimum(a, inv)
    z = r * r
    p = jnp.full_like(r, jnp.float32(_ATAN_COEF[-1]))
    for c in _ATAN_COEF[-2::-1]:
        p = p * z + jnp.float32(c)
    p = p * r
    res = jnp.where(a > jnp.float32(1.0), jnp.float32(0.5) - p, p)
    return jnp.where(d < jnp.float32(0.0), -res, res)


def _loop(n, u=1):
    def deco(body):
        def stepped(i, c):
            for q in range(u):
                body(i * u + q)
            return c
        lax.fori_loop(0, n // u, stepped, 0)
    return deco


def _pair_index(p, lb):
    return ((p >> lb) << (lb + 1)) | (p & ((1 << lb) - 1))


@functools.partial(
    pl.kernel,
    mesh=plsc.VectorSubcoreMesh(core_axis_name="c", subcore_axis_name="s"),
    out_type=jax.ShapeDtypeStruct((_B * _N,), jnp.float32),
    scratch_types=[
        pltpu.VMEM((_WORDS,), jnp.float32),
        pltpu.VMEM((_NLAYER * _WORDS,), jnp.float32),
        pltpu.VMEM((_WORDS,), jnp.float32),
    ],
)
def _sc_topk(vec_hbm, out_hbm, xs, al, vs):
    wid = lax.axis_index("s") * 2 + lax.axis_index("c")
    base = wid * _WORDS
    pltpu.sync_copy(vec_hbm.at[pl.ds(base, _WORDS)], xs)
    lane = lax.broadcasted_iota(jnp.int32, (16,), 0)
    steep = jnp.float32(_STEEP)
    half = jnp.float32(0.5)

    def lane_sign(lj, lk):
        bits = (lane >> lj) & 1
        if lk is not None:
            bits = bits ^ ((lane >> lk) & 1)
        return (1 - 2 * bits).astype(jnp.float32)

    def chunk_sign(w, lk):
        return (1 - 2 * ((w >> (lk - 4)) & 1)).astype(jnp.float32)

    for k, js in _PHASES:
        lk = k.bit_length() - 1
        pair_js = [(j, lyr) for j, lyr in js if j >= 16]
        grp_js = [(j, lyr) for j, lyr in js if j < 16]

        for j, lyr in pair_js:
            lj = j.bit_length() - 1
            lb = lj - 4
            abase = lyr * _WORDS

            @_loop(_CHUNKS // 2, 2)
            def _(p, j=j, lb=lb, lk=lk, abase=abase):
                w = _pair_index(p, lb)
                off = w * 16
                poff = off ^ j
                xi = xs[pl.ds(off, 16)]
                xp = xs[pl.ds(poff, 16)]
                t = _atan_over_pi((xp - xi) * steep)
                alpha = half + chunk_sign(w & 15, lk) * t
                xs[pl.ds(off, 16)] = xp + alpha * (xi - xp)
                xs[pl.ds(poff, 16)] = xi + alpha * (xp - xi)
                al[pl.ds(abase + off, 16)] = alpha
                al[pl.ds(abase + poff, 16)] = alpha

        if grp_js:
            signs = [
                lane_sign(j.bit_length() - 1, lk if lk <= 3 else None)
                for j, _l in grp_js
            ]

            @_loop(_CHUNKS, 2)
            def _(w, grp_js=grp_js, signs=signs, lk=lk):
                off = w * 16
                xv = xs[pl.ds(off, 16)]
                if lk > 3:
                    cs = chunk_sign(w & 15, lk)
                for (j, lyr), sgn in zip(grp_js, signs):
                    xq = _lane_shuffle(xv, lane ^ j)
                    t = _atan_over_pi((xq - xv) * steep)
                    if lk > 3:
                        alpha = half + cs * (sgn * t)
                    else:
                        alpha = half + sgn * t
                    al[pl.ds(lyr * _WORDS + off, 16)] = alpha
                    xv = xq + alpha * (xv - xq)
                xs[pl.ds(off, 16)] = xv

    @_loop(_CHUNKS, 2)
    def _(w):
        off = w * 16
        im = (off + lane) & (_N - 1)
        vs[pl.ds(off, 16)] = jnp.where(
            im >= _N - _KTOP, jnp.float32(1.0), jnp.float32(0.0))

    for k, js in reversed(_PHASES):
        pair_js = [(j, lyr) for j, lyr in js if j >= 16]
        grp_js = [(j, lyr) for j, lyr in js if j < 16]

        if grp_js:
            @_loop(_CHUNKS, 2)
            def _(w, grp_js=grp_js):
                off = w * 16
                vv = vs[pl.ds(off, 16)]
                for j, lyr in reversed(grp_js):
                    a = al[pl.ds(lyr * _WORDS + off, 16)]
                    vq = _lane_shuffle(vv, lane ^ j)
                    vv = vq + a * (vv - vq)
                vs[pl.ds(off, 16)] = vv

        for j, lyr in reversed(pair_js):
            lb = j.bit_length() - 5
            abase = lyr * _WORDS

            @_loop(_CHUNKS // 2, 2)
            def _(p, j=j, lb=lb, abase=abase):
                w = _pair_index(p, lb)
                off = w * 16
                poff = off ^ j
                vi = vs[pl.ds(off, 16)]
                vp = vs[pl.ds(poff, 16)]
                a = al[pl.ds(abase + off, 16)]
                vs[pl.ds(off, 16)] = vp + a * (vi - vp)
                vs[pl.ds(poff, 16)] = vi + a * (vp - vi)

    pltpu.sync_copy(vs, out_hbm.at[pl.ds(base, _WORDS)])


def kernel(vectors):
    out = _sc_topk(vectors.reshape(-1))
    return out.reshape(vectors.shape)


# speedup vs baseline: 54.2277x; 1.4059x over previous
import functools

import jax
import jax.numpy as jnp
from jax import lax
from jax.experimental import pallas as pl
from jax.experimental.pallas import tpu as pltpu
from jax.experimental.pallas import tpu_sc as plsc

_N = 256
_KTOP = 16
_STEEP = 10.0
_B = 128
_NW = 32
_SPW = _B // _NW
_WORDS = _SPW * _N
_CHUNKS = _WORDS // 16
_U = 4


def _phases():
    out = []
    lyr = 0
    k = 2
    while k <= _N:
        js = []
        j = k // 2
        while j >= 1:
            js.append((j, lyr))
            lyr += 1
            j //= 2
        out.append((k, js))
        k *= 2
    return out


_PHASES = _phases()
_NLAYER = sum(len(js) for _, js in _PHASES)

_ATAN_COEF = (0.31827129, -0.10517136, 0.05742714, -0.02718631, 0.0066628)

_GATHER_DNUMS = lax.GatherDimensionNumbers(
    offset_dims=(), collapsed_slice_dims=(0,), start_index_map=(0,))


def _lane_shuffle(vec, idx):
    return lax.gather(
        vec, idx[:, None], _GATHER_DNUMS, slice_sizes=(1,),
        mode=lax.GatherScatterMode.PROMISE_IN_BOUNDS)


def _atan_over_pi(d):
    a = jnp.abs(d)
    inv = jnp.float32(1.0) / jnp.maximum(a, jnp.float32(1e-30))
    r = jnp.minimum(a, inv)
    z = r * r
    p = jnp.full_like(r, jnp.float32(_ATAN_COEF[-1]))
    for c in _ATAN_COEF[-2::-1]:
        p = p * z + jnp.float32(c)
    p = p * r
    res = jnp.where(a > jnp.float32(1.0), jnp.float32(0.5) - p, p)
    return jnp.where(d < jnp.float32(0.0), -res, res)


def _seq(n, stepped):
    lax.fori_loop(0, n, lambda i, c: (stepped(i), c)[1], 0)


def _pair_index(p, lb):
    return ((p >> lb) << (lb + 1)) | (p & ((1 << lb) - 1))


@functools.partial(
    pl.kernel,
    mesh=plsc.VectorSubcoreMesh(core_axis_name="c", subcore_axis_name="s"),
    out_type=jax.ShapeDtypeStruct((_B * _N,), jnp.float32),
    scratch_types=[
        pltpu.VMEM((_WORDS,), jnp.float32),
        pltpu.VMEM((_NLAYER * _WORDS,), jnp.float32),
        pltpu.VMEM((_WORDS,), jnp.float32),
    ],
)
def _sc_topk(vec_hbm, out_hbm, xs, al, vs):
    wid = lax.axis_index("s") * 2 + lax.axis_index("c")
    base = wid * _WORDS
    pltpu.sync_copy(vec_hbm.at[pl.ds(base, _WORDS)], xs)
    lane = lax.broadcasted_iota(jnp.int32, (16,), 0)
    steep = jnp.float32(_STEEP)
    half = jnp.float32(0.5)

    def lane_sign(lj, lk):
        bits = (lane >> lj) & 1
        if lk is not None:
            bits = bits ^ ((lane >> lk) & 1)
        return (1 - 2 * bits).astype(jnp.float32)

    def chunk_sign(w, lk):
        return (1 - 2 * ((w >> (lk - 4)) & 1)).astype(jnp.float32)

    for k, js in _PHASES:
        lk = k.bit_length() - 1
        pair_js = [(j, lyr) for j, lyr in js if j >= 16]
        grp_js = [(j, lyr) for j, lyr in js if j < 16]

        for j, lyr in pair_js:
            lb = j.bit_length() - 5
            abase = lyr * _WORDS

            def stepped(pi, j=j, lb=lb, lk=lk, abase=abase):
                offs, xis, xps = [], [], []
                for q in range(_U):
                    w = _pair_index(pi * _U + q, lb)
                    off = w * 16
                    offs.append((w, off, off ^ j))
                    xis.append(xs[pl.ds(off, 16)])
                    xps.append(xs[pl.ds(off ^ j, 16)])
                for q in range(_U):
                    w, off, poff = offs[q]
                    xi, xp = xis[q], xps[q]
                    t = _atan_over_pi((xp - xi) * steep)
                    alpha = half + chunk_sign(w & 15, lk) * t
                    xs[pl.ds(off, 16)] = xp + alpha * (xi - xp)
                    xs[pl.ds(poff, 16)] = xi + alpha * (xp - xi)
                    al[pl.ds(abase + off, 16)] = alpha
                    al[pl.ds(abase + poff, 16)] = alpha

            _seq(_CHUNKS // 2 // _U, stepped)

        if grp_js:
            signs = [
                lane_sign(j.bit_length() - 1, lk if lk <= 3 else None)
                for j, _l in grp_js
            ]

            def stepped(wi, grp_js=grp_js, signs=signs, lk=lk):
                offs, xvs, css = [], [], []
                for q in range(_U):
                    w = wi * _U + q
                    off = w * 16
                    offs.append(off)
                    xvs.append(xs[pl.ds(off, 16)])
                    css.append(chunk_sign(w & 15, lk) if lk > 3 else None)
                for q in range(_U):
                    off, xv, cs = offs[q], xvs[q], css[q]
                    for (jj, lyr), sgn in zip(grp_js, signs):
                        xq = _lane_shuffle(xv, lane ^ jj)
                        t = _atan_over_pi((xq - xv) * steep)
                        if cs is not None:
                            alpha = half + cs * (sgn * t)
                        else:
                            alpha = half + sgn * t
                        al[pl.ds(lyr * _WORDS + off, 16)] = alpha
                        xv = xq + alpha * (xv - xq)
                    xs[pl.ds(off, 16)] = xv

            _seq(_CHUNKS // _U, stepped)

    def vinit(wi):
        for q in range(_U):
            off = (wi * _U + q) * 16
            im = (off + lane) & (_N - 1)
            vs[pl.ds(off, 16)] = jnp.where(
                im >= _N - _KTOP, jnp.float32(1.0), jnp.float32(0.0))

    _seq(_CHUNKS // _U, vinit)

    for k, js in reversed(_PHASES):
        pair_js = [(j, lyr) for j, lyr in js if j >= 16]
        grp_js = [(j, lyr) for j, lyr in js if j < 16]

        if grp_js:
            def stepped(wi, grp_js=grp_js):
                offs, vvs = [], []
                for q in range(_U):
                    off = (wi * _U + q) * 16
                    offs.append(off)
                    vvs.append(vs[pl.ds(off, 16)])
                for q in range(_U):
                    off, vv = offs[q], vvs[q]
                    for jj, lyr in reversed(grp_js):
                        a = al[pl.ds(lyr * _WORDS + off, 16)]
                        vq = _lane_shuffle(vv, lane ^ jj)
                        vv = vq + a * (vv - vq)
                    vs[pl.ds(off, 16)] = vv

            _seq(_CHUNKS // _U, stepped)

        for j, lyr in reversed(pair_js):
            lb = j.bit_length() - 5
            abase = lyr * _WORDS

            def stepped(pi, j=j, lb=lb, abase=abase):
                offs, vis, vps, als = [], [], [], []
                for q in range(_U):
                    off = _pair_index(pi * _U + q, lb) * 16
                    offs.append((off, off ^ j))
                    vis.append(vs[pl.ds(off, 16)])
                    vps.append(vs[pl.ds(off ^ j, 16)])
                    als.append(al[pl.ds(abase + off, 16)])
                for q in range(_U):
                    off, poff = offs[q]
                    vi, vp, a = vis[q], vps[q], als[q]
                    vs[pl.ds(off, 16)] = vp + a * (vi - vp)
                    vs[pl.ds(poff, 16)] = vi + a * (vp - vi)

            _seq(_CHUNKS // 2 // _U, stepped)

    pltpu.sync_copy(vs, out_hbm.at[pl.ds(base, _WORDS)])


def kernel(vectors):
    out = _sc_topk(vectors.reshape(-1))
    return out.reshape(vectors.shape)


# speedup vs baseline: 72.2436x; 1.3322x over previous
import functools

import jax
import jax.numpy as jnp
from jax import lax
from jax.experimental import pallas as pl
from jax.experimental.pallas import tpu as pltpu
from jax.experimental.pallas import tpu_sc as plsc

_N = 256
_KTOP = 16
_STEEP = 10.0
_B = 128
_NW = 32
_SPW = _B // _NW
_WORDS = _SPW * _N
_CHUNKS = _WORDS // 16
_U = 4
_UP = 8


def _phases():
    out = []
    lyr = 0
    k = 2
    while k <= _N:
        js = []
        j = k // 2
        while j >= 1:
            js.append((j, lyr))
            lyr += 1
            j //= 2
        out.append((k, js))
        k *= 2
    return out


_PHASES = _phases()
_NLAYER = sum(len(js) for _, js in _PHASES)

_ATAN_COEF = (0.31827129, -0.10517136, 0.05742714, -0.02718631, 0.0066628)

_GATHER_DNUMS = lax.GatherDimensionNumbers(
    offset_dims=(), collapsed_slice_dims=(0,), start_index_map=(0,))


def _lane_shuffle(vec, idx):
    return lax.gather(
        vec, idx[:, None], _GATHER_DNUMS, slice_sizes=(1,),
        mode=lax.GatherScatterMode.PROMISE_IN_BOUNDS)


def _signed_atan(d, spos, sneg):
    a = jnp.abs(d)
    inv = jnp.float32(1.0) / a
    r = jnp.minimum(a, inv)
    z = r * r
    p = jnp.full_like(r, jnp.float32(_ATAN_COEF[-1]))
    for c in _ATAN_COEF[-2::-1]:
        p = p * z + jnp.float32(c)
    p = p * r
    res = jnp.where(a > jnp.float32(1.0), jnp.float32(0.5) - p, p)
    return jnp.where(d < jnp.float32(0.0), sneg, spos) * res


def _seq(n, stepped):
    lax.fori_loop(0, n, lambda i, c: (stepped(i), c)[1], 0)


def _pair_index(p, lb):
    return ((p >> lb) << (lb + 1)) | (p & ((1 << lb) - 1))


@functools.partial(
    pl.kernel,
    mesh=plsc.VectorSubcoreMesh(core_axis_name="c", subcore_axis_name="s"),
    out_type=jax.ShapeDtypeStruct((_B * _N,), jnp.float32),
    scratch_types=[
        pltpu.VMEM((_WORDS,), jnp.float32),
        pltpu.VMEM((_NLAYER * _WORDS,), jnp.float32),
        pltpu.VMEM((_WORDS,), jnp.float32),
    ],
)
def _sc_topk(vec_hbm, out_hbm, xs, al, vs):
    wid = lax.axis_index("s") * 2 + lax.axis_index("c")
    base = wid * _WORDS
    pltpu.sync_copy(vec_hbm.at[pl.ds(base, _WORDS)], xs)
    lane = lax.broadcasted_iota(jnp.int32, (16,), 0)
    steep = jnp.float32(_STEEP)
    half = jnp.float32(0.5)

    def lane_sign(lj, lk):
        bits = (lane >> lj) & 1
        if lk is not None:
            bits = bits ^ ((lane >> lk) & 1)
        return (1 - 2 * bits).astype(jnp.float32)

    def chunk_sign(w, lk):
        return (1 - 2 * ((w >> (lk - 4)) & 1)).astype(jnp.float32)

    for k, js in _PHASES:
        lk = k.bit_length() - 1
        pair_js = [(j, lyr) for j, lyr in js if j >= 16]
        grp_js = [(j, lyr) for j, lyr in js if j < 16]

        for j, lyr in pair_js:
            lb = j.bit_length() - 5
            abase = lyr * _WORDS

            def stepped(pi, j=j, lb=lb, lk=lk, abase=abase):
                offs, xis, xps = [], [], []
                for q in range(_UP):
                    w = _pair_index(pi * _UP + q, lb)
                    off = w * 16
                    offs.append((w, off, off ^ j))
                    xis.append(xs[pl.ds(off, 16)])
                    xps.append(xs[pl.ds(off ^ j, 16)])
                for q in range(_UP):
                    w, off, poff = offs[q]
                    xi, xp = xis[q], xps[q]
                    cs = chunk_sign(w & 15, lk)
                    alpha = half + _signed_atan((xp - xi) * steep, cs, -cs)
                    xs[pl.ds(off, 16)] = xp + alpha * (xi - xp)
                    xs[pl.ds(poff, 16)] = xi + alpha * (xp - xi)
                    al[pl.ds(abase + off, 16)] = alpha
                    al[pl.ds(abase + poff, 16)] = alpha

            _seq(_CHUNKS // 2 // _UP, stepped)

        if grp_js:
            signs = [
                lane_sign(j.bit_length() - 1, lk if lk <= 3 else None)
                for j, _l in grp_js
            ]
            nsigns = [-s for s in signs]

            def stepped(wi, grp_js=grp_js, signs=signs, nsigns=nsigns, lk=lk):
                offs, xvs, css = [], [], []
                for q in range(_U):
                    w = wi * _U + q
                    off = w * 16
                    offs.append(off)
                    xvs.append(xs[pl.ds(off, 16)])
                    css.append(chunk_sign(w & 15, lk) if lk > 3 else None)
                for q in range(_U):
                    off, xv, cs = offs[q], xvs[q], css[q]
                    for (jj, lyr), sgn, nsgn in zip(grp_js, signs, nsigns):
                        xq = _lane_shuffle(xv, lane ^ jj)
                        t = _signed_atan((xq - xv) * steep, sgn, nsgn)
                        if cs is not None:
                            alpha = half + cs * t
                        else:
                            alpha = half + t
                        al[pl.ds(lyr * _WORDS + off, 16)] = alpha
                        xv = xq + alpha * (xv - xq)
                    xs[pl.ds(off, 16)] = xv

            _seq(_CHUNKS // _U, stepped)

    def vinit(wi):
        for q in range(_U):
            off = (wi * _U + q) * 16
            im = (off + lane) & (_N - 1)
            vs[pl.ds(off, 16)] = jnp.where(
                im >= _N - _KTOP, jnp.float32(1.0), jnp.float32(0.0))

    _seq(_CHUNKS // _U, vinit)

    for k, js in reversed(_PHASES):
        pair_js = [(j, lyr) for j, lyr in js if j >= 16]
        grp_js = [(j, lyr) for j, lyr in js if j < 16]

        if grp_js:
            def stepped(wi, grp_js=grp_js):
                offs, vvs = [], []
                for q in range(_U):
                    off = (wi * _U + q) * 16
                    offs.append(off)
                    vvs.append(vs[pl.ds(off, 16)])
                for q in range(_U):
                    off, vv = offs[q], vvs[q]
                    for jj, lyr in reversed(grp_js):
                        a = al[pl.ds(lyr * _WORDS + off, 16)]
                        vq = _lane_shuffle(vv, lane ^ jj)
                        vv = vq + a * (vv - vq)
                    vs[pl.ds(off, 16)] = vv

            _seq(_CHUNKS // _U, stepped)

        for j, lyr in reversed(pair_js):
            lb = j.bit_length() - 5
            abase = lyr * _WORDS

            def stepped(pi, j=j, lb=lb, abase=abase):
                offs, vis, vps, als = [], [], [], []
                for q in range(_UP):
                    off = _pair_index(pi * _UP + q, lb) * 16
                    offs.append((off, off ^ j))
                    vis.append(vs[pl.ds(off, 16)])
                    vps.append(vs[pl.ds(off ^ j, 16)])
                    als.append(al[pl.ds(abase + off, 16)])
                for q in range(_UP):
                    off, poff = offs[q]
                    vi, vp, a = vis[q], vps[q], als[q]
                    vs[pl.ds(off, 16)] = vp + a * (vi - vp)
                    vs[pl.ds(poff, 16)] = vi + a * (vp - vi)

            _seq(_CHUNKS // 2 // _UP, stepped)

    pltpu.sync_copy(vs, out_hbm.at[pl.ds(base, _WORDS)])


def kernel(vectors):
    out = _sc_topk(vectors.reshape(-1))
    return out.reshape(vectors.shape)


# speedup vs baseline: 76.5418x; 1.0595x over previous
import functools

import jax
import jax.numpy as jnp
from jax import lax
from jax.experimental import pallas as pl
from jax.experimental.pallas import tpu as pltpu
from jax.experimental.pallas import tpu_sc as plsc

_N = 256
_KTOP = 16
_STEEP = 10.0
_B = 128
_NW = 32
_SPW = _B // _NW
_WORDS = _SPW * _N
_CHUNKS = _WORDS // 16
_U = 4
_UP = 8


def _phases():
    out = []
    lyr = 0
    k = 2
    while k <= _N:
        js = []
        j = k // 2
        while j >= 1:
            js.append((j, lyr))
            lyr += 1
            j //= 2
        out.append((k, js))
        k *= 2
    return out


_PHASES = _phases()
_NLAYER = sum(len(js) for _, js in _PHASES)

_ATAN_COEF = (0.31807679, -0.1023145, 0.0466691, -0.01245679)

_GATHER_DNUMS = lax.GatherDimensionNumbers(
    offset_dims=(), collapsed_slice_dims=(0,), start_index_map=(0,))


def _lane_shuffle(vec, idx):
    return lax.gather(
        vec, idx[:, None], _GATHER_DNUMS, slice_sizes=(1,),
        mode=lax.GatherScatterMode.PROMISE_IN_BOUNDS)


def _signed_atan(d, spos, sneg):
    a = jnp.abs(d)
    inv = jnp.float32(1.0) / a
    r = jnp.minimum(a, inv)
    z = r * r
    p = jnp.full_like(r, jnp.float32(_ATAN_COEF[-1]))
    for c in _ATAN_COEF[-2::-1]:
        p = p * z + jnp.float32(c)
    p = p * r
    res = jnp.where(a > jnp.float32(1.0), jnp.float32(0.5) - p, p)
    return jnp.where(d < jnp.float32(0.0), sneg, spos) * res


def _seq(n, stepped):
    lax.fori_loop(0, n, lambda i, c: (stepped(i), c)[1], 0)


def _pair_index(p, lb):
    return ((p >> lb) << (lb + 1)) | (p & ((1 << lb) - 1))


@functools.partial(
    pl.kernel,
    mesh=plsc.VectorSubcoreMesh(core_axis_name="c", subcore_axis_name="s"),
    out_type=jax.ShapeDtypeStruct((_B, _N), jnp.float32),
    scratch_types=[
        pltpu.VMEM((_WORDS,), jnp.float32),
        pltpu.VMEM((_NLAYER * _WORDS,), jnp.float32),
        pltpu.VMEM((_WORDS,), jnp.float32),
        pltpu.SemaphoreType.DMA,
    ],
)
def _sc_topk(vec_hbm, out_hbm, xs, al, vs, sem):
    wid = lax.axis_index("s") * 2 + lax.axis_index("c")
    row0 = wid * _SPW
    in_cps = [
        pltpu.async_copy(vec_hbm.at[row0 + s], xs.at[pl.ds(s * _N, _N)], sem)
        for s in range(_SPW)
    ]
    for cp in in_cps:
        cp.wait()
    lane = lax.broadcasted_iota(jnp.int32, (16,), 0)
    steep = jnp.float32(_STEEP)
    half = jnp.float32(0.5)

    def lane_sign(lj, lk):
        bits = (lane >> lj) & 1
        if lk is not None:
            bits = bits ^ ((lane >> lk) & 1)
        return (1 - 2 * bits).astype(jnp.float32)

    def chunk_sign(w, lk):
        return (1 - 2 * ((w >> (lk - 4)) & 1)).astype(jnp.float32)

    for k, js in _PHASES:
        lk = k.bit_length() - 1
        pair_js = [(j, lyr) for j, lyr in js if j >= 16]
        grp_js = [(j, lyr) for j, lyr in js if j < 16]

        for j, lyr in pair_js:
            lb = j.bit_length() - 5
            abase = lyr * _WORDS

            def stepped(pi, j=j, lb=lb, lk=lk, abase=abase):
                offs, xis, xps = [], [], []
                for q in range(_UP):
                    w = _pair_index(pi * _UP + q, lb)
                    off = w * 16
                    offs.append((w, off, off ^ j))
                    xis.append(xs[pl.ds(off, 16)])
                    xps.append(xs[pl.ds(off ^ j, 16)])
                for q in range(_UP):
                    w, off, poff = offs[q]
                    xi, xp = xis[q], xps[q]
                    cs = chunk_sign(w & 15, lk)
                    alpha = half + _signed_atan((xp - xi) * steep, cs, -cs)
                    xs[pl.ds(off, 16)] = xp + alpha * (xi - xp)
                    xs[pl.ds(poff, 16)] = xi + alpha * (xp - xi)
                    al[pl.ds(abase + off, 16)] = alpha
                    al[pl.ds(abase + poff, 16)] = alpha

            _seq(_CHUNKS // 2 // _UP, stepped)

        if grp_js:
            signs = [
                lane_sign(j.bit_length() - 1, lk if lk <= 3 else None)
                for j, _l in grp_js
            ]
            nsigns = [-s for s in signs]

            def stepped(wi, grp_js=grp_js, signs=signs, nsigns=nsigns, lk=lk):
                offs, xvs, css = [], [], []
                for q in range(_U):
                    w = wi * _U + q
                    off = w * 16
                    offs.append(off)
                    xvs.append(xs[pl.ds(off, 16)])
                    css.append(chunk_sign(w & 15, lk) if lk > 3 else None)
                for q in range(_U):
                    off, xv, cs = offs[q], xvs[q], css[q]
                    for (jj, lyr), sgn, nsgn in zip(grp_js, signs, nsigns):
                        xq = _lane_shuffle(xv, lane ^ jj)
                        t = _signed_atan((xq - xv) * steep, sgn, nsgn)
                        if cs is not None:
                            alpha = half + cs * t
                        else:
                            alpha = half + t
                        al[pl.ds(lyr * _WORDS + off, 16)] = alpha
                        xv = xq + alpha * (xv - xq)
                    xs[pl.ds(off, 16)] = xv

            _seq(_CHUNKS // _U, stepped)

    def vinit(wi):
        for q in range(_U):
            off = (wi * _U + q) * 16
            im = (off + lane) & (_N - 1)
            vs[pl.ds(off, 16)] = jnp.where(
                im >= _N - _KTOP, jnp.float32(1.0), jnp.float32(0.0))

    _seq(_CHUNKS // _U, vinit)

    for k, js in reversed(_PHASES):
        pair_js = [(j, lyr) for j, lyr in js if j >= 16]
        grp_js = [(j, lyr) for j, lyr in js if j < 16]

        if grp_js:
            def stepped(wi, grp_js=grp_js):
                offs, vvs = [], []
                for q in range(_U):
                    off = (wi * _U + q) * 16
                    offs.append(off)
                    vvs.append(vs[pl.ds(off, 16)])
                for q in range(_U):
                    off, vv = offs[q], vvs[q]
                    for jj, lyr in reversed(grp_js):
                        a = al[pl.ds(lyr * _WORDS + off, 16)]
                        vq = _lane_shuffle(vv, lane ^ jj)
                        vv = vq + a * (vv - vq)
                    vs[pl.ds(off, 16)] = vv

            _seq(_CHUNKS // _U, stepped)

        for j, lyr in reversed(pair_js):
            lb = j.bit_length() - 5
            abase = lyr * _WORDS

            def stepped(pi, j=j, lb=lb, abase=abase):
                offs, vis, vps, als = [], [], [], []
                for q in range(_UP):
                    off = _pair_index(pi * _UP + q, lb) * 16
                    offs.append((off, off ^ j))
                    vis.append(vs[pl.ds(off, 16)])
                    vps.append(vs[pl.ds(off ^ j, 16)])
                    als.append(al[pl.ds(abase + off, 16)])
                for q in range(_UP):
                    off, poff = offs[q]
                    vi, vp, a = vis[q], vps[q], als[q]
                    vs[pl.ds(off, 16)] = vp + a * (vi - vp)
                    vs[pl.ds(poff, 16)] = vi + a * (vp - vi)

            _seq(_CHUNKS // 2 // _UP, stepped)

    out_cps = [
        pltpu.async_copy(vs.at[pl.ds(s * _N, _N)], out_hbm.at[row0 + s], sem)
        for s in range(_SPW)
    ]
    for cp in out_cps:
        cp.wait()


def kernel(vectors):
    return _sc_topk(vectors)


# speedup vs baseline: 81.1772x; 1.0606x over previous
import functools

import jax
import jax.numpy as jnp
from jax import lax
from jax.experimental import pallas as pl
from jax.experimental.pallas import tpu as pltpu
from jax.experimental.pallas import tpu_sc as plsc

_N = 256
_KTOP = 16
_STEEP = 10.0
_B = 128
_NW = 32
_SPW = _B // _NW
_WORDS = _SPW * _N
_CHUNKS = _WORDS // 16
_U = 8
_UP = 8


def _phases():
    out = []
    lyr = 0
    k = 2
    while k <= _N:
        js = []
        j = k // 2
        while j >= 1:
            js.append((j, lyr))
            lyr += 1
            j //= 2
        out.append((k, js))
        k *= 2
    return out


_PHASES = _phases()
_NLAYER = sum(len(js) for _, js in _PHASES)

_ATAN_COEF = (0.31807679, -0.1023145, 0.0466691, -0.01245679)

_GATHER_DNUMS = lax.GatherDimensionNumbers(
    offset_dims=(), collapsed_slice_dims=(0,), start_index_map=(0,))


def _lane_shuffle(vec, idx):
    return lax.gather(
        vec, idx[:, None], _GATHER_DNUMS, slice_sizes=(1,),
        mode=lax.GatherScatterMode.PROMISE_IN_BOUNDS)


def _signed_atan(d, spos, sneg):
    a = jnp.abs(d)
    inv = jnp.float32(1.0) / a
    r = jnp.minimum(a, inv)
    z = r * r
    p = jnp.full_like(r, jnp.float32(_ATAN_COEF[-1]))
    for c in _ATAN_COEF[-2::-1]:
        p = p * z + jnp.float32(c)
    p = p * r
    res = jnp.where(a > jnp.float32(1.0), jnp.float32(0.5) - p, p)
    return jnp.where(d < jnp.float32(0.0), sneg, spos) * res


def _seq(n, stepped):
    lax.fori_loop(0, n, lambda i, c: (stepped(i), c)[1], 0)


def _pair_index(p, lb):
    return ((p >> lb) << (lb + 1)) | (p & ((1 << lb) - 1))


@functools.partial(
    pl.kernel,
    mesh=plsc.VectorSubcoreMesh(core_axis_name="c", subcore_axis_name="s"),
    out_type=jax.ShapeDtypeStruct((_B, _N), jnp.float32),
    scratch_types=[
        pltpu.VMEM((_WORDS,), jnp.float32),
        pltpu.VMEM((_NLAYER * _WORDS,), jnp.float32),
        pltpu.VMEM((_WORDS,), jnp.float32),
        pltpu.SemaphoreType.DMA,
    ],
)
def _sc_topk(vec_hbm, out_hbm, xs, al, vs, sem):
    wid = lax.axis_index("s") * 2 + lax.axis_index("c")
    row0 = wid * _SPW
    in_cps = [
        pltpu.async_copy(vec_hbm.at[row0 + s], xs.at[pl.ds(s * _N, _N)], sem)
        for s in range(_SPW)
    ]
    for cp in in_cps:
        cp.wait()
    lane = lax.broadcasted_iota(jnp.int32, (16,), 0)
    steep = jnp.float32(_STEEP)
    half = jnp.float32(0.5)

    def lane_sign(lj, lk):
        bits = (lane >> lj) & 1
        if lk is not None:
            bits = bits ^ ((lane >> lk) & 1)
        return (1 - 2 * bits).astype(jnp.float32)

    def chunk_sign(w, lk):
        return (1 - 2 * ((w >> (lk - 4)) & 1)).astype(jnp.float32)

    for k, js in _PHASES:
        lk = k.bit_length() - 1
        pair_js = [(j, lyr) for j, lyr in js if j >= 16]
        grp_js = [(j, lyr) for j, lyr in js if j < 16]

        for j, lyr in pair_js:
            lb = j.bit_length() - 5
            abase = lyr * _WORDS

            def stepped(pi, j=j, lb=lb, lk=lk, abase=abase):
                offs, xis, xps = [], [], []
                for q in range(_UP):
                    w = _pair_index(pi * _UP + q, lb)
                    off = w * 16
                    offs.append((w, off, off ^ j))
                    xis.append(xs[pl.ds(off, 16)])
                    xps.append(xs[pl.ds(off ^ j, 16)])
                for q in range(_UP):
                    w, off, poff = offs[q]
                    xi, xp = xis[q], xps[q]
                    cs = chunk_sign(w & 15, lk)
                    alpha = half + _signed_atan((xp - xi) * steep, cs, -cs)
                    xs[pl.ds(off, 16)] = xp + alpha * (xi - xp)
                    xs[pl.ds(poff, 16)] = xi + alpha * (xp - xi)
                    al[pl.ds(abase + off, 16)] = alpha
                    al[pl.ds(abase + poff, 16)] = alpha

            _seq(_CHUNKS // 2 // _UP, stepped)

        if grp_js:
            signs = [
                lane_sign(j.bit_length() - 1, lk if lk <= 3 else None)
                for j, _l in grp_js
            ]
            nsigns = [-s for s in signs]

            def stepped(wi, grp_js=grp_js, signs=signs, nsigns=nsigns, lk=lk):
                offs, xvs, css = [], [], []
                for q in range(_U):
                    w = wi * _U + q
                    off = w * 16
                    offs.append(off)
                    xvs.append(xs[pl.ds(off, 16)])
                    css.append(chunk_sign(w & 15, lk) if lk > 3 else None)
                for q in range(_U):
                    off, xv, cs = offs[q], xvs[q], css[q]
                    for (jj, lyr), sgn, nsgn in zip(grp_js, signs, nsigns):
                        xq = _lane_shuffle(xv, lane ^ jj)
                        t = _signed_atan((xq - xv) * steep, sgn, nsgn)
                        if cs is not None:
                            alpha = half + cs * t
                        else:
                            alpha = half + t
                        al[pl.ds(lyr * _WORDS + off, 16)] = alpha
                        xv = xq + alpha * (xv - xq)
                    xs[pl.ds(off, 16)] = xv

            _seq(_CHUNKS // _U, stepped)

    def vinit(wi):
        for q in range(_U):
            off = (wi * _U + q) * 16
            im = (off + lane) & (_N - 1)
            vs[pl.ds(off, 16)] = jnp.where(
                im >= _N - _KTOP, jnp.float32(1.0), jnp.float32(0.0))

    _seq(_CHUNKS // _U, vinit)

    for k, js in reversed(_PHASES):
        pair_js = [(j, lyr) for j, lyr in js if j >= 16]
        grp_js = [(j, lyr) for j, lyr in js if j < 16]

        if grp_js:
            def stepped(wi, grp_js=grp_js):
                offs, vvs = [], []
                for q in range(_U):
                    off = (wi * _U + q) * 16
                    offs.append(off)
                    vvs.append(vs[pl.ds(off, 16)])
                for q in range(_U):
                    off, vv = offs[q], vvs[q]
                    for jj, lyr in reversed(grp_js):
                        a = al[pl.ds(lyr * _WORDS + off, 16)]
                        vq = _lane_shuffle(vv, lane ^ jj)
                        vv = vq + a * (vv - vq)
                    vs[pl.ds(off, 16)] = vv

            _seq(_CHUNKS // _U, stepped)

        for j, lyr in reversed(pair_js):
            lb = j.bit_length() - 5
            abase = lyr * _WORDS

            def stepped(pi, j=j, lb=lb, abase=abase):
                offs, vis, vps, als = [], [], [], []
                for q in range(_UP):
                    off = _pair_index(pi * _UP + q, lb) * 16
                    offs.append((off, off ^ j))
                    vis.append(vs[pl.ds(off, 16)])
                    vps.append(vs[pl.ds(off ^ j, 16)])
                    als.append(al[pl.ds(abase + off, 16)])
                for q in range(_UP):
                    off, poff = offs[q]
                    vi, vp, a = vis[q], vps[q], als[q]
                    vs[pl.ds(off, 16)] = vp + a * (vi - vp)
                    vs[pl.ds(poff, 16)] = vi + a * (vp - vi)

            _seq(_CHUNKS // 2 // _UP, stepped)

    out_cps = [
        pltpu.async_copy(vs.at[pl.ds(s * _N, _N)], out_hbm.at[row0 + s], sem)
        for s in range(_SPW)
    ]
    for cp in out_cps:
        cp.wait()


def kernel(vectors):
    return _sc_topk(vectors)


# speedup vs baseline: 81.3642x; 1.0023x over previous
import functools

import jax
import jax.numpy as jnp
from jax import lax
from jax.experimental import pallas as pl
from jax.experimental.pallas import tpu as pltpu
from jax.experimental.pallas import tpu_sc as plsc

_N = 256
_KTOP = 16
_STEEP = 10.0
_B = 128
_NW = 32
_SPW = _B // _NW
_WORDS = _SPW * _N
_CHUNKS = _WORDS // 16
_U = 8
_UP = 16


def _phases():
    out = []
    lyr = 0
    k = 2
    while k <= _N:
        js = []
        j = k // 2
        while j >= 1:
            js.append((j, lyr))
            lyr += 1
            j //= 2
        out.append((k, js))
        k *= 2
    return out


_PHASES = _phases()
_NLAYER = sum(len(js) for _, js in _PHASES)

_ATAN_COEF = (0.31807679, -0.1023145, 0.0466691, -0.01245679)

_GATHER_DNUMS = lax.GatherDimensionNumbers(
    offset_dims=(), collapsed_slice_dims=(0,), start_index_map=(0,))


def _lane_shuffle(vec, idx):
    return lax.gather(
        vec, idx[:, None], _GATHER_DNUMS, slice_sizes=(1,),
        mode=lax.GatherScatterMode.PROMISE_IN_BOUNDS)


def _signed_atan(d, spos, sneg):
    a = jnp.abs(d)
    inv = jnp.float32(1.0) / a
    r = jnp.minimum(a, inv)
    z = r * r
    p = jnp.full_like(r, jnp.float32(_ATAN_COEF[-1]))
    for c in _ATAN_COEF[-2::-1]:
        p = p * z + jnp.float32(c)
    p = p * r
    res = jnp.where(a > jnp.float32(1.0), jnp.float32(0.5) - p, p)
    return jnp.where(d < jnp.float32(0.0), sneg, spos) * res


def _seq(n, stepped):
    lax.fori_loop(0, n, lambda i, c: (stepped(i), c)[1], 0)


def _pair_index(p, lb):
    return ((p >> lb) << (lb + 1)) | (p & ((1 << lb) - 1))


@functools.partial(
    pl.kernel,
    mesh=plsc.VectorSubcoreMesh(core_axis_name="c", subcore_axis_name="s"),
    out_type=jax.ShapeDtypeStruct((_B, _N), jnp.float32),
    scratch_types=[
        pltpu.VMEM((_WORDS,), jnp.float32),
        pltpu.VMEM((_NLAYER * _WORDS,), jnp.float32),
        pltpu.VMEM((_WORDS,), jnp.float32),
        pltpu.SemaphoreType.DMA,
    ],
)
def _sc_topk(vec_hbm, out_hbm, xs, al, vs, sem):
    wid = lax.axis_index("s") * 2 + lax.axis_index("c")
    row0 = wid * _SPW
    in_cps = [
        pltpu.async_copy(vec_hbm.at[row0 + s], xs.at[pl.ds(s * _N, _N)], sem)
        for s in range(_SPW)
    ]
    for cp in in_cps:
        cp.wait()
    lane = lax.broadcasted_iota(jnp.int32, (16,), 0)
    steep = jnp.float32(_STEEP)
    half = jnp.float32(0.5)

    def lane_sign(lj, lk):
        bits = (lane >> lj) & 1
        if lk is not None:
            bits = bits ^ ((lane >> lk) & 1)
        return (1 - 2 * bits).astype(jnp.float32)

    def chunk_sign(w, lk):
        return (1 - 2 * ((w >> (lk - 4)) & 1)).astype(jnp.float32)

    for k, js in _PHASES:
        lk = k.bit_length() - 1
        pair_js = [(j, lyr) for j, lyr in js if j >= 16]
        grp_js = [(j, lyr) for j, lyr in js if j < 16]

        for j, lyr in pair_js:
            lb = j.bit_length() - 5
            abase = lyr * _WORDS

            def stepped(pi, j=j, lb=lb, lk=lk, abase=abase):
                offs, xis, xps = [], [], []
                for q in range(_UP):
                    w = _pair_index(pi * _UP + q, lb)
                    off = w * 16
                    offs.append((w, off, off ^ j))
                    xis.append(xs[pl.ds(off, 16)])
                    xps.append(xs[pl.ds(off ^ j, 16)])
                for q in range(_UP):
                    w, off, poff = offs[q]
                    xi, xp = xis[q], xps[q]
                    cs = chunk_sign(w & 15, lk)
                    alpha = half + _signed_atan((xp - xi) * steep, cs, -cs)
                    xs[pl.ds(off, 16)] = xp + alpha * (xi - xp)
                    xs[pl.ds(poff, 16)] = xi + alpha * (xp - xi)
                    al[pl.ds(abase + off, 16)] = alpha
                    al[pl.ds(abase + poff, 16)] = alpha

            _seq(_CHUNKS // 2 // _UP, stepped)

        if grp_js:
            signs = [
                lane_sign(j.bit_length() - 1, lk if lk <= 3 else None)
                for j, _l in grp_js
            ]
            nsigns = [-s for s in signs]

            def stepped(wi, grp_js=grp_js, signs=signs, nsigns=nsigns, lk=lk):
                offs, xvs, css = [], [], []
                for q in range(_U):
                    w = wi * _U + q
                    off = w * 16
                    offs.append(off)
                    xvs.append(xs[pl.ds(off, 16)])
                    css.append(chunk_sign(w & 15, lk) if lk > 3 else None)
                for q in range(_U):
                    off, xv, cs = offs[q], xvs[q], css[q]
                    for (jj, lyr), sgn, nsgn in zip(grp_js, signs, nsigns):
                        xq = _lane_shuffle(xv, lane ^ jj)
                        t = _signed_atan((xq - xv) * steep, sgn, nsgn)
                        if cs is not None:
                            alpha = half + cs * t
                        else:
                            alpha = half + t
                        al[pl.ds(lyr * _WORDS + off, 16)] = alpha
                        xv = xq + alpha * (xv - xq)
                    xs[pl.ds(off, 16)] = xv

            _seq(_CHUNKS // _U, stepped)

    def vinit(wi):
        for q in range(_U):
            off = (wi * _U + q) * 16
            im = (off + lane) & (_N - 1)
            vs[pl.ds(off, 16)] = jnp.where(
                im >= _N - _KTOP, jnp.float32(1.0), jnp.float32(0.0))

    _seq(_CHUNKS // _U, vinit)

    for k, js in reversed(_PHASES):
        pair_js = [(j, lyr) for j, lyr in js if j >= 16]
        grp_js = [(j, lyr) for j, lyr in js if j < 16]

        if grp_js:
            def stepped(wi, grp_js=grp_js):
                offs, vvs = [], []
                for q in range(_U):
                    off = (wi * _U + q) * 16
                    offs.append(off)
                    vvs.append(vs[pl.ds(off, 16)])
                for q in range(_U):
                    off, vv = offs[q], vvs[q]
                    for jj, lyr in reversed(grp_js):
                        a = al[pl.ds(lyr * _WORDS + off, 16)]
                        vq = _lane_shuffle(vv, lane ^ jj)
                        vv = vq + a * (vv - vq)
                    vs[pl.ds(off, 16)] = vv

            _seq(_CHUNKS // _U, stepped)

        for j, lyr in reversed(pair_js):
            lb = j.bit_length() - 5
            abase = lyr * _WORDS

            def stepped(pi, j=j, lb=lb, abase=abase):
                offs, vis, vps, als = [], [], [], []
                for q in range(_UP):
                    off = _pair_index(pi * _UP + q, lb) * 16
                    offs.append((off, off ^ j))
                    vis.append(vs[pl.ds(off, 16)])
                    vps.append(vs[pl.ds(off ^ j, 16)])
                    als.append(al[pl.ds(abase + off, 16)])
                for q in range(_UP):
                    off, poff = offs[q]
                    vi, vp, a = vis[q], vps[q], als[q]
                    vs[pl.ds(off, 16)] = vp + a * (vi - vp)
                    vs[pl.ds(poff, 16)] = vi + a * (vp - vi)

            _seq(_CHUNKS // 2 // _UP, stepped)

    out_cps = [
        pltpu.async_copy(vs.at[pl.ds(s * _N, _N)], out_hbm.at[row0 + s], sem)
        for s in range(_SPW)
    ]
    for cp in out_cps:
        cp.wait()


def kernel(vectors):
    return _sc_topk(vectors)
